# Initial kernel scaffold; baseline (speedup 1.0000x reference)
import jax
import jax.numpy as jnp
from jax.experimental import pallas as pl


def kernel(x, edge_index, edge_attr, W1, b1, W2, b2, lw1, lb1, lw2, lb2):
    raise NotImplementedError("write your pallas kernel here")


# speedup vs baseline: 8.3086x; 8.3086x over previous
import functools

import jax
import jax.numpy as jnp
from jax import lax
from jax.experimental import pallas as pl
from jax.experimental.pallas import tpu as pltpu
from jax.experimental.pallas import tpu_sc as plsc

NC = 2
NS = 16
NW = NC * NS
CH = 128
LANES = 16


def _deg_kernel(npad, k):
    mesh = plsc.VectorSubcoreMesh(core_axis_name="c", subcore_axis_name="s")
    per_tile = npad // NS

    @functools.partial(
        pl.kernel,
        out_type=jax.ShapeDtypeStruct((NC, npad), jnp.float32),
        mesh=mesh,
        scratch_types=[
            pltpu.VMEM((CH,), jnp.int32),
            pltpu.VMEM((CH,), jnp.float32),
            pltpu.VMEM((per_tile,), jnp.float32),
            pltpu.VMEM_SHARED((npad,), jnp.float32),
        ],
    )
    def deg_kernel(dst_hbm, ew_hbm, out_hbm, idx_v, ew_v, z_v, deg_sp):
        cid = lax.axis_index("c")
        sid = lax.axis_index("s")
        wid = sid * NC + cid
        zv = jnp.zeros((LANES,), jnp.float32)

        def zero_body(i, _):
            z_v[pl.ds(i * LANES, LANES)] = zv
            return 0

        lax.fori_loop(0, per_tile // LANES, zero_body, 0)
        pltpu.sync_copy(z_v, deg_sp.at[pl.ds(sid * per_tile, per_tile)])
        plsc.subcore_barrier()

        def body(j, _):
            pltpu.sync_copy(dst_hbm.at[wid, j], idx_v)
            pltpu.sync_copy(ew_hbm.at[wid, j], ew_v)
            pltpu.sync_copy(ew_v, deg_sp.at[idx_v], add=True)
            return 0

        lax.fori_loop(0, k, body, 0)
        plsc.subcore_barrier()
        pltpu.sync_copy(
            deg_sp.at[pl.ds(sid * per_tile, per_tile)],
            out_hbm.at[cid, pl.ds(sid * per_tile, per_tile)],
        )

    return deg_kernel


def _agg_kernel(npad, f, k):
    mesh = plsc.VectorSubcoreMesh(core_axis_name="c", subcore_axis_name="s")
    rows_per_tile = npad // NS
    n_full = rows_per_tile // CH
    rem = rows_per_tile % CH
    cols = f // LANES

    @functools.partial(
        pl.kernel,
        out_type=jax.ShapeDtypeStruct((NC, npad, f), jnp.float32),
        mesh=mesh,
        scratch_types=[
            pltpu.VMEM((CH,), jnp.int32),
            pltpu.VMEM((CH,), jnp.int32),
            pltpu.VMEM((CH,), jnp.float32),
            pltpu.VMEM((CH, f), jnp.float32),
            pltpu.SemaphoreType.DMA,
            pltpu.VMEM_SHARED((npad, f), jnp.float32),
        ],
    )
    def agg_kernel(hs_hbm, src_hbm, dst_hbm, ew_hbm, out_hbm,
                   srcv, dstv, ewv, rows, sem, acc_sp):
        cid = lax.axis_index("c")
        sid = lax.axis_index("s")
        wid = sid * NC + cid
        zv = jnp.zeros((LANES,), jnp.float32)

        def zero_row(i, _):
            for c in range(cols):
                rows[i, pl.ds(c * LANES, LANES)] = zv
            return 0

        lax.fori_loop(0, CH, zero_row, 0)
        base = sid * rows_per_tile
        for r0 in range(n_full):
            pltpu.sync_copy(rows, acc_sp.at[pl.ds(base + r0 * CH, CH)])
        if rem:
            pltpu.sync_copy(rows.at[pl.ds(0, rem)],
                            acc_sp.at[pl.ds(base + n_full * CH, rem)])
        plsc.subcore_barrier()

        def body(j, _):
            pltpu.sync_copy(src_hbm.at[wid, j], srcv)
            pltpu.sync_copy(ew_hbm.at[wid, j], ewv)
            pltpu.async_copy(hs_hbm.at[srcv], rows, sem).wait()
            pltpu.sync_copy(dst_hbm.at[wid, j], dstv)

            def scale(b, _):
                ew16 = ewv[pl.ds(b * LANES, LANES)]
                for e in range(LANES):
                    w = ew16[e]
                    i = b * LANES + e
                    for c in range(cols):
                        sl = pl.ds(c * LANES, LANES)
                        rows[i, sl] = rows[i, sl] * w
                return 0

            lax.fori_loop(0, CH // LANES, scale, 0)
            pltpu.sync_copy(rows, acc_sp.at[dstv], add=True)
            return 0

        lax.fori_loop(0, k, body, 0)
        plsc.subcore_barrier()
        pltpu.sync_copy(acc_sp.at[pl.ds(base, rows_per_tile)],
                        out_hbm.at[cid, pl.ds(base, rows_per_tile)])

    return agg_kernel


def _tc_a(deg3, x, w1, block):
    n, f = x.shape

    def body(deg_ref, x_ref, w_ref, hs_ref, dinv_ref):
        deg = deg_ref[0] + deg_ref[1] + 1.0
        dinv = lax.rsqrt(deg)
        dinv = dinv * (1.5 - 0.5 * deg * dinv * dinv)
        h = jnp.dot(x_ref[...], w_ref[...], preferred_element_type=jnp.float32)
        hs_ref[...] = h * dinv
        dinv_ref[...] = dinv

    return pl.pallas_call(
        body,
        grid=(n // block,),
        in_specs=[
            pl.BlockSpec((2, block, 1), lambda i: (0, i, 0)),
            pl.BlockSpec((block, f), lambda i: (i, 0)),
            pl.BlockSpec((f, f), lambda i: (0, 0)),
        ],
        out_specs=[
            pl.BlockSpec((block, f), lambda i: (i, 0)),
            pl.BlockSpec((block, 1), lambda i: (i, 0)),
        ],
        out_shape=[
            jax.ShapeDtypeStruct((n, f), jnp.float32),
            jax.ShapeDtypeStruct((n, 1), jnp.float32),
        ],
    )(deg3, x, w1)


def _tc_b(acc, hs, dinv, b1, w2, block):
    n, f = hs.shape

    def body(acc_ref, hs_ref, dinv_ref, b_ref, w_ref, out_ref):
        a = acc_ref[0] + acc_ref[1] + hs_ref[...]
        o1 = jnp.maximum(a * dinv_ref[...] + b_ref[...], 0.0)
        out_ref[...] = jnp.dot(
            o1, w_ref[...], preferred_element_type=jnp.float32) * dinv_ref[...]

    return pl.pallas_call(
        body,
        grid=(n // block,),
        in_specs=[
            pl.BlockSpec((2, block, f), lambda i: (0, i, 0)),
            pl.BlockSpec((block, f), lambda i: (i, 0)),
            pl.BlockSpec((block, 1), lambda i: (i, 0)),
            pl.BlockSpec((1, f), lambda i: (0, 0)),
            pl.BlockSpec((f, f), lambda i: (0, 0)),
        ],
        out_specs=pl.BlockSpec((block, f), lambda i: (i, 0)),
        out_shape=jax.ShapeDtypeStruct((n, f), jnp.float32),
    )(acc, hs, dinv, b1, w2)


def _tc_c(acc, hs, dinv, b2, lw1p, lb1p, lw2p, lb2, block):
    n, f = hs.shape

    def body(acc_ref, hs_ref, dinv_ref, b_ref, lw1_ref, lb1_ref, lw2_ref,
             lb2_ref, out_ref, accum):
        i = pl.program_id(0)

        @pl.when(i == 0)
        def _():
            accum[...] = jnp.zeros_like(accum)

        a = acc_ref[0] + acc_ref[1] + hs_ref[...]
        o2 = jnp.maximum(a * dinv_ref[...] + b_ref[...], 0.0)
        accum[...] += jnp.sum(o2, axis=0, keepdims=True)

        @pl.when(i == pl.num_programs(0) - 1)
        def _():
            g = accum[...] * jnp.float32(1.0 / n)
            t = jnp.maximum(
                jnp.dot(g, lw1_ref[...], preferred_element_type=jnp.float32)
                + lb1_ref[...], 0.0)
            out_ref[...] = (
                jnp.dot(t, lw2_ref[...], preferred_element_type=jnp.float32)
                + lb2_ref[...])

    return pl.pallas_call(
        body,
        grid=(n // block,),
        in_specs=[
            pl.BlockSpec((2, block, f), lambda i: (0, i, 0)),
            pl.BlockSpec((block, f), lambda i: (i, 0)),
            pl.BlockSpec((block, 1), lambda i: (i, 0)),
            pl.BlockSpec((1, f), lambda i: (0, 0)),
            pl.BlockSpec((f, f), lambda i: (0, 0)),
            pl.BlockSpec((1, f), lambda i: (0, 0)),
            pl.BlockSpec((f, 1), lambda i: (0, 0)),
            pl.BlockSpec((1, 1), lambda i: (0, 0)),
        ],
        out_specs=pl.BlockSpec((1, 1), lambda i: (0, 0)),
        out_shape=jax.ShapeDtypeStruct((1, 1), jnp.float32),
        scratch_shapes=[pltpu.VMEM((1, f), jnp.float32)],
    )(acc, hs, dinv, b2, lw1p, lb1p, lw2p, lb2)


def kernel(x, edge_index, edge_attr, W1, b1, W2, b2, lw1, lb1, lw2, lb2):
    n, f = x.shape
    e = edge_attr.shape[0]
    h = lw1.shape[1]

    per_w = -(-e // NW)
    k = -(-per_w // CH)
    epad = NW * k * CH
    pad = epad - e

    src = jnp.pad(edge_index[0], (0, pad)).reshape(NW, k, CH)
    dst = jnp.pad(edge_index[1], (0, pad)).reshape(NW, k, CH)
    ewp = jnp.pad(edge_attr, (0, pad)).reshape(NW, k, CH)

    npad = NS * (-(-n // (NS * LANES))) * LANES
    degp = _deg_kernel(npad, k)(dst, ewp)
    deg3 = degp[:, :n].reshape(2, n, 1)

    block = 2000
    hs1, dinv = _tc_a(deg3, x, W1, block)

    agg = _agg_kernel(npad, f, k)
    acc1 = agg(hs1, src, dst, ewp)[:, :n]
    hs2 = _tc_b(acc1, hs1, dinv, b1.reshape(1, f), W2, block)

    acc2 = agg(hs2, src, dst, ewp)[:, :n]

    lw1p = jnp.pad(lw1, ((0, 0), (0, f - h)))
    lb1p = jnp.pad(lb1, (0, f - h)).reshape(1, f)
    lw2p = jnp.pad(lw2, ((0, f - h), (0, 0)))
    out = _tc_c(acc2, hs2, dinv, b2.reshape(1, f), lw1p, lb1p, lw2p,
                lb2.reshape(1, 1), block)
    return out.reshape(1)


# speedup vs baseline: 14.3970x; 1.7328x over previous
import functools

import jax
import jax.numpy as jnp
from jax import lax
from jax.experimental import pallas as pl
from jax.experimental.pallas import tpu as pltpu
from jax.experimental.pallas import tpu_sc as plsc

NC = 2
NS = 16
NW = NC * NS
CH = 128
LANES = 16


def _deg_kernel(npad, k):
    mesh = plsc.VectorSubcoreMesh(core_axis_name="c", subcore_axis_name="s")
    per_tile = npad // NS

    @functools.partial(
        pl.kernel,
        out_type=jax.ShapeDtypeStruct((NC, npad), jnp.float32),
        mesh=mesh,
        scratch_types=[
            pltpu.VMEM((k, CH), jnp.int32),
            pltpu.VMEM((k, CH), jnp.float32),
            pltpu.VMEM((per_tile,), jnp.float32),
            pltpu.SemaphoreType.DMA,
            pltpu.VMEM_SHARED((npad,), jnp.float32),
        ],
    )
    def deg_kernel(dst_hbm, ew_hbm, out_hbm, idx_v, ew_v, z_v, sem, deg_sp):
        cid = lax.axis_index("c")
        sid = lax.axis_index("s")
        wid = sid * NC + cid
        zv = jnp.zeros((LANES,), jnp.float32)

        def zero_body(i, _):
            z_v[pl.ds(i * LANES, LANES)] = zv
            return 0

        lax.fori_loop(0, per_tile // LANES, zero_body, 0)
        pltpu.sync_copy(z_v, deg_sp.at[pl.ds(sid * per_tile, per_tile)])
        pltpu.sync_copy(dst_hbm.at[wid], idx_v)
        pltpu.sync_copy(ew_hbm.at[wid], ew_v)
        plsc.subcore_barrier()

        def body(j, _):
            pltpu.async_copy(ew_v.at[j], deg_sp.at[idx_v.at[j]], sem,
                             add=True)
            return 0

        lax.fori_loop(0, k, body, 0)

        def drain(j, _):
            pltpu.make_async_copy(ew_v.at[j], deg_sp.at[idx_v.at[j]],
                                  sem).wait()
            return 0

        lax.fori_loop(0, k, drain, 0)
        plsc.subcore_barrier()
        pltpu.sync_copy(
            deg_sp.at[pl.ds(sid * per_tile, per_tile)],
            out_hbm.at[cid, pl.ds(sid * per_tile, per_tile)],
        )

    return deg_kernel


def _agg_kernel(npad, f, k):
    mesh = plsc.VectorSubcoreMesh(core_axis_name="c", subcore_axis_name="s")
    rows_per_tile = npad // NS
    n_full = rows_per_tile // CH
    rem = rows_per_tile % CH
    cols = f // LANES

    @functools.partial(
        pl.kernel,
        out_type=jax.ShapeDtypeStruct((NC, npad, f), jnp.float32),
        mesh=mesh,
        scratch_types=[
            pltpu.VMEM((4, 2, CH), jnp.int32),
            pltpu.VMEM((4, CH), jnp.float32),
            pltpu.VMEM((CH, f), jnp.float32),
            pltpu.VMEM((CH, f), jnp.float32),
            pltpu.SemaphoreType.DMA,
            pltpu.SemaphoreType.DMA,
            pltpu.SemaphoreType.DMA,
            pltpu.SemaphoreType.DMA,
            [pltpu.SemaphoreType.DMA] * 4,
            pltpu.VMEM_SHARED((npad, f), jnp.float32),
        ],
    )
    def agg_kernel(hs_hbm, e2_hbm, ew_hbm, out_hbm, ring, ewr, rows0,
                   rows1, gsem0, gsem1, ssem0, ssem1, isems, acc_sp):
        cid = lax.axis_index("c")
        sid = lax.axis_index("s")
        wid = sid * NC + cid
        bufs = ((rows0, gsem0, ssem0), (rows1, gsem1, ssem1))
        zv = jnp.zeros((LANES,), jnp.float32)

        def zero_row(i, _):
            for c in range(cols):
                rows0[i, pl.ds(c * LANES, LANES)] = zv
            return 0

        lax.fori_loop(0, CH, zero_row, 0)
        base = sid * rows_per_tile
        for r0 in range(n_full):
            pltpu.sync_copy(rows0, acc_sp.at[pl.ds(base + r0 * CH, CH)])
        if rem:
            pltpu.sync_copy(rows0.at[pl.ds(0, rem)],
                            acc_sp.at[pl.ds(base + n_full * CH, rem)])
        plsc.subcore_barrier()

        pltpu.async_copy(e2_hbm.at[wid, 0], ring.at[0], isems[0])
        pltpu.async_copy(ew_hbm.at[wid, 0], ewr.at[0], isems[0])
        if k > 1:
            pltpu.async_copy(e2_hbm.at[wid, 1], ring.at[1], isems[1])
            pltpu.async_copy(ew_hbm.at[wid, 1], ewr.at[1], isems[1])
        pltpu.make_async_copy(e2_hbm.at[wid, 0], ring.at[0], isems[0]).wait()
        pltpu.make_async_copy(ew_hbm.at[wid, 0], ewr.at[0], isems[0]).wait()
        pltpu.async_copy(hs_hbm.at[ring.at[0, 0]], rows0, gsem0)

        def quad_body(j4, _):
            for b in range(4):
                j = j4 * 4 + b
                rb, gs, ss = bufs[b % 2]
                ro, go, so = bufs[1 - b % 2]
                s_cur = b % 4
                s_nxt = (b + 1) % 4
                s_pre = (b + 2) % 4

                @pl.when(j < k)
                def _():
                    @pl.when(j >= 1)
                    def _():
                        pltpu.make_async_copy(
                            ro, acc_sp.at[ring.at[s_nxt, 1]], so).wait()

                    @pl.when(j + 2 < k)
                    def _():
                        pltpu.async_copy(e2_hbm.at[wid, j + 2],
                                         ring.at[s_pre], isems[s_pre])
                        pltpu.async_copy(ew_hbm.at[wid, j + 2],
                                         ewr.at[s_pre], isems[s_pre])

                    @pl.when(j + 1 < k)
                    def _():
                        pltpu.make_async_copy(e2_hbm.at[wid, j + 1],
                                              ring.at[s_nxt],
                                              isems[s_nxt]).wait()
                        pltpu.make_async_copy(ew_hbm.at[wid, j + 1],
                                              ewr.at[s_nxt],
                                              isems[s_nxt]).wait()
                        pltpu.async_copy(hs_hbm.at[ring.at[s_nxt, 0]], ro,
                                         go)

                    pltpu.make_async_copy(hs_hbm.at[ring.at[s_cur, 0]], rb,
                                          gs).wait()

                    def scale(blk, _):
                        ew16 = ewr[s_cur, pl.ds(blk * LANES, LANES)]
                        for e in range(LANES):
                            w = ew16[e]
                            i = blk * LANES + e
                            for c in range(cols):
                                sl = pl.ds(c * LANES, LANES)
                                rb[i, sl] = rb[i, sl] * w
                        return 0

                    lax.fori_loop(0, CH // LANES, scale, 0)
                    pltpu.async_copy(rb, acc_sp.at[ring.at[s_cur, 1]], ss,
                                     add=True)
            return 0

        lax.fori_loop(0, (k + 3) // 4, quad_body, 0)
        rl, _, sl = bufs[(k - 1) % 2]
        pltpu.make_async_copy(rl, acc_sp.at[ring.at[(k - 1) % 4, 1]],
                              sl).wait()
        plsc.subcore_barrier()
        pltpu.sync_copy(acc_sp.at[pl.ds(base, rows_per_tile)],
                        out_hbm.at[cid, pl.ds(base, rows_per_tile)])

    return agg_kernel


def _tc_a(deg3, x, w1, block):
    n, f = x.shape

    def body(deg_ref, x_ref, w_ref, hs_ref, dinv_ref):
        deg = deg_ref[0] + deg_ref[1] + 1.0
        dinv = lax.rsqrt(deg)
        dinv = dinv * (1.5 - 0.5 * deg * dinv * dinv)
        h = jnp.dot(x_ref[...], w_ref[...], preferred_element_type=jnp.float32)
        hs_ref[...] = h * dinv
        dinv_ref[...] = dinv

    return pl.pallas_call(
        body,
        grid=(n // block,),
        in_specs=[
            pl.BlockSpec((2, block, 1), lambda i: (0, i, 0)),
            pl.BlockSpec((block, f), lambda i: (i, 0)),
            pl.BlockSpec((f, f), lambda i: (0, 0)),
        ],
        out_specs=[
            pl.BlockSpec((block, f), lambda i: (i, 0)),
            pl.BlockSpec((block, 1), lambda i: (i, 0)),
        ],
        out_shape=[
            jax.ShapeDtypeStruct((n, f), jnp.float32),
            jax.ShapeDtypeStruct((n, 1), jnp.float32),
        ],
    )(deg3, x, w1)


def _tc_b(acc, hs, dinv, b1, w2, block):
    n, f = hs.shape

    def body(acc_ref, hs_ref, dinv_ref, b_ref, w_ref, out_ref):
        a = acc_ref[0] + acc_ref[1] + hs_ref[...]
        o1 = jnp.maximum(a * dinv_ref[...] + b_ref[...], 0.0)
        out_ref[...] = jnp.dot(
            o1, w_ref[...], preferred_element_type=jnp.float32) * dinv_ref[...]

    return pl.pallas_call(
        body,
        grid=(n // block,),
        in_specs=[
            pl.BlockSpec((2, block, f), lambda i: (0, i, 0)),
            pl.BlockSpec((block, f), lambda i: (i, 0)),
            pl.BlockSpec((block, 1), lambda i: (i, 0)),
            pl.BlockSpec((1, f), lambda i: (0, 0)),
            pl.BlockSpec((f, f), lambda i: (0, 0)),
        ],
        out_specs=pl.BlockSpec((block, f), lambda i: (i, 0)),
        out_shape=jax.ShapeDtypeStruct((n, f), jnp.float32),
    )(acc, hs, dinv, b1, w2)


def _tc_c(acc, hs, dinv, b2, lw1p, lb1p, lw2p, lb2, block):
    n, f = hs.shape

    def body(acc_ref, hs_ref, dinv_ref, b_ref, lw1_ref, lb1_ref, lw2_ref,
             lb2_ref, out_ref, accum):
        i = pl.program_id(0)

        @pl.when(i == 0)
        def _():
            accum[...] = jnp.zeros_like(accum)

        a = acc_ref[0] + acc_ref[1] + hs_ref[...]
        o2 = jnp.maximum(a * dinv_ref[...] + b_ref[...], 0.0)
        accum[...] += jnp.sum(o2, axis=0, keepdims=True)

        @pl.when(i == pl.num_programs(0) - 1)
        def _():
            g = accum[...] * jnp.float32(1.0 / n)
            t = jnp.maximum(
                jnp.dot(g, lw1_ref[...], preferred_element_type=jnp.float32)
                + lb1_ref[...], 0.0)
            out_ref[...] = (
                jnp.dot(t, lw2_ref[...], preferred_element_type=jnp.float32)
                + lb2_ref[...])

    return pl.pallas_call(
        body,
        grid=(n // block,),
        in_specs=[
            pl.BlockSpec((2, block, f), lambda i: (0, i, 0)),
            pl.BlockSpec((block, f), lambda i: (i, 0)),
            pl.BlockSpec((block, 1), lambda i: (i, 0)),
            pl.BlockSpec((1, f), lambda i: (0, 0)),
            pl.BlockSpec((f, f), lambda i: (0, 0)),
            pl.BlockSpec((1, f), lambda i: (0, 0)),
            pl.BlockSpec((f, 1), lambda i: (0, 0)),
            pl.BlockSpec((1, 1), lambda i: (0, 0)),
        ],
        out_specs=pl.BlockSpec((1, 1), lambda i: (0, 0)),
        out_shape=jax.ShapeDtypeStruct((1, 1), jnp.float32),
        scratch_shapes=[pltpu.VMEM((1, f), jnp.float32)],
    )(acc, hs, dinv, b2, lw1p, lb1p, lw2p, lb2)


def kernel(x, edge_index, edge_attr, W1, b1, W2, b2, lw1, lb1, lw2, lb2):
    n, f = x.shape
    e = edge_attr.shape[0]
    h = lw1.shape[1]

    per_w = -(-e // NW)
    k = -(-per_w // CH)
    epad = NW * k * CH
    pad = epad - e

    src = jnp.pad(edge_index[0], (0, pad)).reshape(NW, k, CH)
    dst = jnp.pad(edge_index[1], (0, pad)).reshape(NW, k, CH)
    ewp = jnp.pad(edge_attr, (0, pad)).reshape(NW, k, CH)
    e2 = jnp.stack([src, dst], axis=2)

    npad = NS * (-(-n // (NS * LANES))) * LANES
    degp = _deg_kernel(npad, k)(dst, ewp)
    deg3 = degp[:, :n].reshape(2, n, 1)

    block = 2000
    hs1, dinv = _tc_a(deg3, x, W1, block)

    agg = _agg_kernel(npad, f, k)
    acc1 = agg(hs1, e2, ewp)[:, :n]
    hs2 = _tc_b(acc1, hs1, dinv, b1.reshape(1, f), W2, block)

    acc2 = agg(hs2, e2, ewp)[:, :n]

    lw1p = jnp.pad(lw1, ((0, 0), (0, f - h)))
    lb1p = jnp.pad(lb1, (0, f - h)).reshape(1, f)
    lw2p = jnp.pad(lw2, ((0, f - h), (0, 0)))
    out = _tc_c(acc2, hs2, dinv, b2.reshape(1, f), lw1p, lb1p, lw2p,
                lb2.reshape(1, 1), block)
    return out.reshape(1)


# speedup vs baseline: 27.3584x; 1.9003x over previous
import functools

import jax
import jax.numpy as jnp
from jax import lax
from jax.experimental import pallas as pl
from jax.experimental.pallas import tpu as pltpu
from jax.experimental.pallas import tpu_sc as plsc

NC = 2
NS = 16
NW = NC * NS
CH = 80
LANES = 16


def _deg_kernel(npad, k):
    mesh = plsc.VectorSubcoreMesh(core_axis_name="c", subcore_axis_name="s")
    per_tile = npad // NS

    @functools.partial(
        pl.kernel,
        out_type=jax.ShapeDtypeStruct((NC, npad), jnp.float32),
        mesh=mesh,
        scratch_types=[
            pltpu.VMEM((k, CH), jnp.int32),
            pltpu.VMEM((k, CH), jnp.float32),
            pltpu.VMEM((per_tile,), jnp.float32),
            pltpu.SemaphoreType.DMA,
            pltpu.VMEM_SHARED((npad,), jnp.float32),
        ],
    )
    def deg_kernel(dst_hbm, ew_hbm, out_hbm, idx_v, ew_v, z_v, sem, deg_sp):
        cid = lax.axis_index("c")
        sid = lax.axis_index("s")
        wid = sid * NC + cid
        zv = jnp.zeros((LANES,), jnp.float32)

        def zero_body(i, _):
            z_v[pl.ds(i * LANES, LANES)] = zv
            return 0

        lax.fori_loop(0, per_tile // LANES, zero_body, 0)
        pltpu.sync_copy(z_v, deg_sp.at[pl.ds(sid * per_tile, per_tile)])
        pltpu.sync_copy(dst_hbm.at[wid], idx_v)
        pltpu.sync_copy(ew_hbm.at[wid], ew_v)
        plsc.subcore_barrier()

        def body(j, _):
            pltpu.async_copy(ew_v.at[j], deg_sp.at[idx_v.at[j]], sem,
                             add=True)
            return 0

        lax.fori_loop(0, k, body, 0)

        def drain(j, _):
            pltpu.make_async_copy(ew_v.at[j], deg_sp.at[idx_v.at[j]],
                                  sem).wait()
            return 0

        lax.fori_loop(0, k, drain, 0)
        plsc.subcore_barrier()
        pltpu.sync_copy(
            deg_sp.at[pl.ds(sid * per_tile, per_tile)],
            out_hbm.at[cid, pl.ds(sid * per_tile, per_tile)],
        )

    return deg_kernel


def _agg_kernel(npad, f, k):
    mesh = plsc.VectorSubcoreMesh(core_axis_name="c", subcore_axis_name="s")
    rows_per_tile = npad // NS
    n_full = rows_per_tile // CH
    rem = rows_per_tile % CH
    cols = f // LANES

    @functools.partial(
        pl.kernel,
        out_type=jax.ShapeDtypeStruct((NC, npad, f), jnp.float32),
        mesh=mesh,
        scratch_types=[
            pltpu.VMEM((8, 2, CH), jnp.int32),
            pltpu.VMEM((8, CH), jnp.float32),
            [pltpu.VMEM((CH, f), jnp.float32)] * 4,
            [pltpu.SemaphoreType.DMA] * 4,
            [pltpu.SemaphoreType.DMA] * 4,
            [pltpu.SemaphoreType.DMA] * 8,
            pltpu.VMEM_SHARED((npad, f), jnp.float32),
        ],
    )
    def agg_kernel(hs_hbm, e2_hbm, ew_hbm, out_hbm, ring, ewr, rows,
                   gsems, ssems, isems, acc_sp):
        cid = lax.axis_index("c")
        sid = lax.axis_index("s")
        wid = sid * NC + cid
        zv = jnp.zeros((LANES,), jnp.float32)

        def zero_row(i, _):
            for c in range(cols):
                rows[0][i, pl.ds(c * LANES, LANES)] = zv
            return 0

        lax.fori_loop(0, CH, zero_row, 0)
        base = sid * rows_per_tile
        for r0 in range(n_full):
            pltpu.sync_copy(rows[0], acc_sp.at[pl.ds(base + r0 * CH, CH)])
        if rem:
            pltpu.sync_copy(rows[0].at[pl.ds(0, rem)],
                            acc_sp.at[pl.ds(base + n_full * CH, rem)])
        plsc.subcore_barrier()

        for m in range(min(4, k)):
            pltpu.async_copy(e2_hbm.at[wid, m], ring.at[m], isems[m])
            pltpu.async_copy(ew_hbm.at[wid, m], ewr.at[m], isems[m])
        for m in range(min(2, k)):
            pltpu.make_async_copy(e2_hbm.at[wid, m], ring.at[m],
                                  isems[m]).wait()
            pltpu.make_async_copy(ew_hbm.at[wid, m], ewr.at[m],
                                  isems[m]).wait()
            pltpu.async_copy(hs_hbm.at[ring.at[m, 0]], rows[m], gsems[m])

        def oct_body(j8, _):
            for b in range(8):
                j = j8 * 8 + b
                rb, gs, ss = rows[b % 4], gsems[b % 4], ssems[b % 4]
                s_cur = b
                s_g = (b + 2) % 8
                s_p = (b + 4) % 8
                b_g = (b + 2) % 4

                @pl.when(j < k)
                def _():
                    pltpu.make_async_copy(hs_hbm.at[ring.at[s_cur, 0]], rb,
                                          gs).wait()

                    def scale(blk, _):
                        ew16 = ewr[s_cur, pl.ds(blk * LANES, LANES)]
                        for e in range(LANES):
                            w = ew16[e]
                            i = blk * LANES + e
                            for c in range(cols):
                                sl = pl.ds(c * LANES, LANES)
                                rb[i, sl] = rb[i, sl] * w
                        return 0

                    lax.fori_loop(0, CH // LANES, scale, 0)
                    pltpu.async_copy(rb, acc_sp.at[ring.at[s_cur, 1]], ss,
                                     add=True)

                    @pl.when(j >= 2)
                    def _():
                        pltpu.make_async_copy(
                            rows[b_g], acc_sp.at[ring.at[s_g, 1]],
                            ssems[b_g]).wait()

                    @pl.when(j + 2 < k)
                    def _():
                        pltpu.make_async_copy(e2_hbm.at[wid, j + 2],
                                              ring.at[s_g],
                                              isems[s_g]).wait()
                        pltpu.make_async_copy(ew_hbm.at[wid, j + 2],
                                              ewr.at[s_g],
                                              isems[s_g]).wait()
                        pltpu.async_copy(hs_hbm.at[ring.at[s_g, 0]],
                                         rows[b_g], gsems[b_g])

                    @pl.when(j + 4 < k)
                    def _():
                        pltpu.async_copy(e2_hbm.at[wid, j + 4],
                                         ring.at[s_p], isems[s_p])
                        pltpu.async_copy(ew_hbm.at[wid, j + 4],
                                         ewr.at[s_p], isems[s_p])
            return 0

        lax.fori_loop(0, (k + 7) // 8, oct_body, 0)
        for m in range(max(0, k - 2), k):
            pltpu.make_async_copy(rows[m % 4], acc_sp.at[ring.at[m % 8, 1]],
                                  ssems[m % 4]).wait()
        plsc.subcore_barrier()
        pltpu.sync_copy(acc_sp.at[pl.ds(base, rows_per_tile)],
                        out_hbm.at[cid, pl.ds(base, rows_per_tile)])

    return agg_kernel


def _tc_a(deg3, x, w1, block):
    n, f = x.shape

    def body(deg_ref, x_ref, w_ref, hs_ref, dinv_ref):
        deg = deg_ref[0] + deg_ref[1] + 1.0
        dinv = lax.rsqrt(deg)
        dinv = dinv * (1.5 - 0.5 * deg * dinv * dinv)
        h = jnp.dot(x_ref[...], w_ref[...], preferred_element_type=jnp.float32)
        hs_ref[...] = h * dinv
        dinv_ref[...] = dinv

    return pl.pallas_call(
        body,
        grid=(n // block,),
        in_specs=[
            pl.BlockSpec((2, block, 1), lambda i: (0, i, 0)),
            pl.BlockSpec((block, f), lambda i: (i, 0)),
            pl.BlockSpec((f, f), lambda i: (0, 0)),
        ],
        out_specs=[
            pl.BlockSpec((block, f), lambda i: (i, 0)),
            pl.BlockSpec((block, 1), lambda i: (i, 0)),
        ],
        out_shape=[
            jax.ShapeDtypeStruct((n, f), jnp.float32),
            jax.ShapeDtypeStruct((n, 1), jnp.float32),
        ],
    )(deg3, x, w1)


def _tc_b(acc, hs, dinv, b1, w2, block):
    n, f = hs.shape

    def body(acc_ref, hs_ref, dinv_ref, b_ref, w_ref, out_ref):
        a = acc_ref[0] + acc_ref[1] + hs_ref[...]
        o1 = jnp.maximum(a * dinv_ref[...] + b_ref[...], 0.0)
        out_ref[...] = jnp.dot(
            o1, w_ref[...], preferred_element_type=jnp.float32) * dinv_ref[...]

    return pl.pallas_call(
        body,
        grid=(n // block,),
        in_specs=[
            pl.BlockSpec((2, block, f), lambda i: (0, i, 0)),
            pl.BlockSpec((block, f), lambda i: (i, 0)),
            pl.BlockSpec((block, 1), lambda i: (i, 0)),
            pl.BlockSpec((1, f), lambda i: (0, 0)),
            pl.BlockSpec((f, f), lambda i: (0, 0)),
        ],
        out_specs=pl.BlockSpec((block, f), lambda i: (i, 0)),
        out_shape=jax.ShapeDtypeStruct((n, f), jnp.float32),
    )(acc, hs, dinv, b1, w2)


def _tc_c(acc, hs, dinv, b2, lw1p, lb1p, lw2p, lb2, block):
    n, f = hs.shape

    def body(acc_ref, hs_ref, dinv_ref, b_ref, lw1_ref, lb1_ref, lw2_ref,
             lb2_ref, out_ref, accum):
        i = pl.program_id(0)

        @pl.when(i == 0)
        def _():
            accum[...] = jnp.zeros_like(accum)

        a = acc_ref[0] + acc_ref[1] + hs_ref[...]
        o2 = jnp.maximum(a * dinv_ref[...] + b_ref[...], 0.0)
        accum[...] += jnp.sum(o2, axis=0, keepdims=True)

        @pl.when(i == pl.num_programs(0) - 1)
        def _():
            g = accum[...] * jnp.float32(1.0 / n)
            t = jnp.maximum(
                jnp.dot(g, lw1_ref[...], preferred_element_type=jnp.float32)
                + lb1_ref[...], 0.0)
            out_ref[...] = (
                jnp.sum(t * lw2_ref[...], axis=1, keepdims=True)
                + lb2_ref[...])

    return pl.pallas_call(
        body,
        grid=(n // block,),
        in_specs=[
            pl.BlockSpec((2, block, f), lambda i: (0, i, 0)),
            pl.BlockSpec((block, f), lambda i: (i, 0)),
            pl.BlockSpec((block, 1), lambda i: (i, 0)),
            pl.BlockSpec((1, f), lambda i: (0, 0)),
            pl.BlockSpec((f, f), lambda i: (0, 0)),
            pl.BlockSpec((1, f), lambda i: (0, 0)),
            pl.BlockSpec((1, f), lambda i: (0, 0)),
            pl.BlockSpec((1, 1), lambda i: (0, 0)),
        ],
        out_specs=pl.BlockSpec((1, 1), lambda i: (0, 0)),
        out_shape=jax.ShapeDtypeStruct((1, 1), jnp.float32),
        scratch_shapes=[pltpu.VMEM((1, f), jnp.float32)],
    )(acc, hs, dinv, b2, lw1p, lb1p, lw2p, lb2)


def kernel(x, edge_index, edge_attr, W1, b1, W2, b2, lw1, lb1, lw2, lb2):
    n, f = x.shape
    e = edge_attr.shape[0]
    h = lw1.shape[1]

    per_w = -(-e // NW)
    k = -(-per_w // CH)
    epad = NW * k * CH
    pad = epad - e

    src = jnp.pad(edge_index[0], (0, pad)).reshape(NW, k, CH)
    dst = jnp.pad(edge_index[1], (0, pad)).reshape(NW, k, CH)
    ewp = jnp.pad(edge_attr, (0, pad)).reshape(NW, k, CH)
    e2 = jnp.stack([src, dst], axis=2)

    npad = NS * (-(-n // (NS * LANES))) * LANES
    degp = _deg_kernel(npad, k)(dst, ewp)
    deg3 = degp[:, :n].reshape(2, n, 1)

    block = 2000
    hs1, dinv = _tc_a(deg3, x, W1, block)

    agg = _agg_kernel(npad, f, k)
    acc1 = agg(hs1, e2, ewp)[:, :n]
    hs2 = _tc_b(acc1, hs1, dinv, b1.reshape(1, f), W2, block)

    acc2 = agg(hs2, e2, ewp)[:, :n]

    lw1p = jnp.pad(lw1, ((0, 0), (0, f - h)))
    lb1p = jnp.pad(lb1, (0, f - h)).reshape(1, f)
    lw2p = jnp.pad(lw2, ((0, f - h), (0, 0))).reshape(1, f)
    out = _tc_c(acc2, hs2, dinv, b2.reshape(1, f), lw1p, lb1p, lw2p,
                lb2.reshape(1, 1), block)
    return out.reshape(1)
